# baseline (speedup 1.0000x reference)
import functools
import math

import jax
import jax.numpy as jnp
from jax import lax
from jax.experimental import pallas as pl
from jax.experimental.pallas import tpu as pltpu
from jax.experimental.pallas import tpu_sc as plsc

_LANES = 16
_CHUNK_ROWS = 32


def _make_sc_kernel(batch, seq_len, d_model, scale):
    info = plsc.get_sparse_core_info()
    nw = info.num_cores * info.num_subcores
    total_rows = batch * seq_len
    rows_per_w = total_rows // nw
    assert total_rows % nw == 0
    assert rows_per_w % _CHUNK_ROWS == 0
    assert seq_len % rows_per_w == 0
    chunk = _CHUNK_ROWS * d_model
    n_chunks = rows_per_w // _CHUNK_ROWS
    mesh = plsc.VectorSubcoreMesh(core_axis_name="c", subcore_axis_name="s")

    @functools.partial(
        pl.kernel,
        out_type=jax.ShapeDtypeStruct((total_rows * d_model,), jnp.float32),
        mesh=mesh,
        scratch_types=[
            pltpu.VMEM((chunk,), jnp.float32),
            pltpu.VMEM((chunk,), jnp.float32),
            pltpu.SemaphoreType.DMA,
        ],
    )
    def sc_kernel(x_hbm, pe_hbm, o_hbm, xbuf, pebuf, sem):
        wid = lax.axis_index("s") * info.num_cores + lax.axis_index("c")
        x_base = wid * (rows_per_w * d_model)
        pe_base = (wid * rows_per_w % seq_len) * d_model

        def step(c, _):
            xoff = x_base + c * chunk
            poff = pe_base + c * chunk
            cp_x = pltpu.async_copy(x_hbm.at[pl.ds(xoff, chunk)], xbuf, sem)
            cp_pe = pltpu.async_copy(pe_hbm.at[pl.ds(poff, chunk)], pebuf, sem)
            cp_x.wait()
            cp_pe.wait()

            def body(i, _):
                sl = pl.ds(i * _LANES, _LANES)
                xbuf[sl] = xbuf[sl] + pebuf[sl] * scale
                return ()

            lax.fori_loop(0, chunk // _LANES, body, (), unroll=4)
            pltpu.sync_copy(xbuf, o_hbm.at[pl.ds(xoff, chunk)])
            return ()

        lax.fori_loop(0, n_chunks, step, ())

    return sc_kernel


def kernel(x, pos_embed):
    batch, seq_len, d_model = x.shape
    scale = math.sqrt(d_model)
    pe = pos_embed[:seq_len].reshape(-1)
    xf = x.reshape(-1)
    sc = _make_sc_kernel(batch, seq_len, d_model, scale)
    out = sc(xf, pe)
    return out.reshape(x.shape)


# speedup vs baseline: 8.4409x; 8.4409x over previous
import math

import jax
import jax.numpy as jnp
from jax.experimental import pallas as pl


_BS = 1024


def _pe_add_kernel(x_ref, pe_ref, o_ref, *, scale):
    o_ref[...] = x_ref[...] + pe_ref[...] * scale


def kernel(x, pos_embed):
    batch, seq_len, d_model = x.shape
    scale = math.sqrt(d_model)
    pe = pos_embed[:seq_len]

    bs = min(_BS, seq_len)
    grid = (seq_len // bs, batch)

    return pl.pallas_call(
        lambda xr, pr, orf: _pe_add_kernel(xr, pr, orf, scale=scale),
        grid=grid,
        in_specs=[
            pl.BlockSpec((1, bs, d_model), lambda s, b: (b, s, 0)),
            pl.BlockSpec((bs, d_model), lambda s, b: (s, 0)),
        ],
        out_specs=pl.BlockSpec((1, bs, d_model), lambda s, b: (b, s, 0)),
        out_shape=jax.ShapeDtypeStruct(x.shape, x.dtype),
    )(x, pe)


# speedup vs baseline: 8.7845x; 1.0407x over previous
import math

import jax
import jax.numpy as jnp
from jax.experimental import pallas as pl


_BS = 2048


def _pe_add_kernel(x_ref, pe_ref, o_ref, *, scale):
    o_ref[...] = x_ref[...] + pe_ref[...] * scale


def kernel(x, pos_embed):
    batch, seq_len, d_model = x.shape
    scale = math.sqrt(d_model)
    pe = pos_embed[:seq_len]

    bs = min(_BS, seq_len)
    grid = (seq_len // bs, batch)

    return pl.pallas_call(
        lambda xr, pr, orf: _pe_add_kernel(xr, pr, orf, scale=scale),
        grid=grid,
        in_specs=[
            pl.BlockSpec((1, bs, d_model), lambda s, b: (b, s, 0)),
            pl.BlockSpec((bs, d_model), lambda s, b: (s, 0)),
        ],
        out_specs=pl.BlockSpec((1, bs, d_model), lambda s, b: (b, s, 0)),
        out_shape=jax.ShapeDtypeStruct(x.shape, x.dtype),
    )(x, pe)
